# baseline (speedup 1.0000x reference)
import functools

import jax
import jax.numpy as jnp
from jax import lax
from jax.experimental import pallas as pl
from jax.experimental.pallas import tpu as pltpu
from jax.experimental.pallas import tpu_sc as plsc

DIM = 128
VOCAB = 100
CHUNK = 128
NBUF = 4
UNROLL = 16
LANES = 16


def _project_body(emb_ref, w_ref, b_ref, out_ref):
    out_ref[...] = lax.dot_general(
        emb_ref[...], w_ref[...],
        dimension_numbers=(((1,), (1,)), ((), ())),
        preferred_element_type=jnp.float32,
    ) + b_ref[...]


def _project_table(emb_table, W, b):
    return pl.pallas_call(
        _project_body,
        out_shape=jax.ShapeDtypeStruct((VOCAB, DIM), jnp.float32),
    )(emb_table, W, b.reshape(1, DIM))


def _make_sc_gather(n_rows):
    info = plsc.get_sparse_core_info()
    nc, ns = info.num_cores, info.num_subcores
    nw = nc * ns
    assert n_rows % (nw * CHUNK * NBUF) == 0
    per_w = n_rows // nw
    n_chunks = per_w // CHUNK
    n_iters = n_chunks // NBUF
    mesh = plsc.VectorSubcoreMesh(core_axis_name="c", subcore_axis_name="s")

    scratch = (
        [pltpu.VMEM((VOCAB * DIM,), jnp.float32)]
        + [pltpu.VMEM((CHUNK,), jnp.int32) for _ in range(NBUF)]
        + [pltpu.VMEM((CHUNK * DIM,), jnp.float32) for _ in range(NBUF)]
        + [pltpu.SemaphoreType.DMA] * (2 * NBUF + 1)
    )

    @functools.partial(
        pl.kernel,
        mesh=mesh,
        out_type=jax.ShapeDtypeStruct((n_rows * DIM,), jnp.float32),
        scratch_types=scratch,
        compiler_params=pltpu.CompilerParams(needs_layout_passes=False),
    )
    def sc_gather(table_hbm, idx_hbm, out_hbm, *bufs):
        table_v = bufs[0]
        idx_v = bufs[1:1 + NBUF]
        rows_v = bufs[1 + NBUF:1 + 2 * NBUF]
        idx_sem = bufs[1 + 2 * NBUF:1 + 3 * NBUF]
        out_sem = bufs[1 + 3 * NBUF:1 + 4 * NBUF]
        tab_sem = bufs[1 + 4 * NBUF]
        wid = lax.axis_index("s") * nc + lax.axis_index("c")
        base = wid * per_w

        def idx_copy(g, b):
            off = base + g * CHUNK
            return pltpu.make_async_copy(
                idx_hbm.at[pl.ds(off, CHUNK)], idx_v[b], idx_sem[b])

        def out_copy(g, b):
            off = (base + g * CHUNK) * DIM
            return pltpu.make_async_copy(
                rows_v[b], out_hbm.at[pl.ds(off, CHUNK * DIM)], out_sem[b])

        pltpu.make_async_copy(table_hbm, table_v, tab_sem).start()
        for b in range(NBUF):
            idx_copy(b, b).start()
        pltpu.make_async_copy(table_hbm, table_v, tab_sem).wait()

        dst_stride = lax.iota(jnp.int32, LANES) * DIM

        def assemble(b):
            idx_ref = idx_v[b]
            rows_ref = rows_v[b]

            def rows_body(u, carry):
                r0 = u * UNROLL
                ivec = idx_ref[pl.ds(r0, UNROLL)]
                src_base = ivec * DIM
                dst_base = dst_stride + r0 * DIM
                for j in range(DIM):
                    col = plsc.load_gather(table_v, [src_base + j])
                    plsc.store_scatter(rows_ref, [dst_base + j], col)
                return carry

            lax.fori_loop(0, CHUNK // UNROLL, rows_body, 0)

        def body(j, carry):
            g0 = j * NBUF
            for b in range(NBUF):
                idx_copy(g0 + b, b).wait()

                @pl.when(j > 0)
                def _(b=b):
                    out_copy(g0 + b - NBUF, b).wait()

                assemble(b)
                out_copy(g0 + b, b).start()

                @pl.when(j < n_iters - 1)
                def _(b=b):
                    idx_copy(g0 + b + NBUF, b).start()
            return carry

        lax.fori_loop(0, n_iters, body, 0)
        for b in range(NBUF):
            out_copy(n_chunks - NBUF + b, b).wait()

    return sc_gather


def kernel(x, emb_table, W, b):
    batch, hist = x.shape
    table_proj = _project_table(emb_table, W, b)
    flat_idx = x.reshape(-1)
    gather = _make_sc_gather(batch * hist)
    out = gather(table_proj.reshape(-1), flat_idx)
    return out.reshape(batch, hist, DIM)


# speedup vs baseline: 5.0621x; 5.0621x over previous
import functools

import jax
import jax.numpy as jnp
from jax import lax
from jax.experimental import pallas as pl
from jax.experimental.pallas import tpu as pltpu
from jax.experimental.pallas import tpu_sc as plsc

DIM = 128
VOCAB = 100
CHUNK = 128
NBUF = 4
UNROLL = 16
LANES = 16


def _project_body(emb_ref, w_ref, b_ref, out_ref):
    out_ref[...] = lax.dot_general(
        emb_ref[...], w_ref[...],
        dimension_numbers=(((1,), (1,)), ((), ())),
        preferred_element_type=jnp.float32,
    ) + b_ref[...]


def _project_table(emb_table, W, b):
    return pl.pallas_call(
        _project_body,
        out_shape=jax.ShapeDtypeStruct((VOCAB, DIM), jnp.float32),
    )(emb_table, W, b.reshape(1, DIM))


def _make_sc_gather(n_rows):
    info = plsc.get_sparse_core_info()
    nc, ns = info.num_cores, info.num_subcores
    nw = nc * ns
    assert n_rows % (nw * CHUNK * NBUF) == 0
    per_w = n_rows // nw
    n_chunks = per_w // CHUNK
    n_iters = n_chunks // NBUF
    mesh = plsc.VectorSubcoreMesh(core_axis_name="c", subcore_axis_name="s")

    scratch = (
        [pltpu.VMEM((VOCAB * DIM,), jnp.float32)]
        + [pltpu.VMEM((CHUNK,), jnp.int32) for _ in range(NBUF)]
        + [pltpu.VMEM((CHUNK * DIM,), jnp.float32) for _ in range(NBUF)]
        + [pltpu.SemaphoreType.DMA] * (2 * NBUF + 1)
    )

    @functools.partial(
        pl.kernel,
        mesh=mesh,
        out_type=jax.ShapeDtypeStruct((n_rows * DIM,), jnp.float32),
        scratch_types=scratch,
        compiler_params=pltpu.CompilerParams(needs_layout_passes=False),
    )
    def sc_gather(table_hbm, idx_hbm, out_hbm, *bufs):
        table_v = bufs[0]
        idx_v = bufs[1:1 + NBUF]
        rows_v = bufs[1 + NBUF:1 + 2 * NBUF]
        idx_sem = bufs[1 + 2 * NBUF:1 + 3 * NBUF]
        out_sem = bufs[1 + 3 * NBUF:1 + 4 * NBUF]
        tab_sem = bufs[1 + 4 * NBUF]
        wid = lax.axis_index("s") * nc + lax.axis_index("c")
        base = wid * per_w

        def idx_copy(g, b):
            off = base + g * CHUNK
            return pltpu.make_async_copy(
                idx_hbm.at[pl.ds(off, CHUNK)], idx_v[b], idx_sem[b])

        def out_copy(g, b):
            off = (base + g * CHUNK) * DIM
            return pltpu.make_async_copy(
                rows_v[b], out_hbm.at[pl.ds(off, CHUNK * DIM)], out_sem[b])

        pltpu.make_async_copy(table_hbm, table_v, tab_sem).start()
        for b in range(NBUF):
            idx_copy(b, b).start()
        pltpu.make_async_copy(table_hbm, table_v, tab_sem).wait()

        lane_iota = lax.iota(jnp.int32, LANES)

        def assemble(b):
            idx_ref = idx_v[b]
            rows_ref = rows_v[b]

            def rows_body(u, carry):
                r0 = u * UNROLL
                ivec = idx_ref[pl.ds(r0, UNROLL)]
                src_rows = ivec * DIM
                for k in range(UNROLL):
                    kvec = jnp.full((LANES,), k, jnp.int32)
                    base = jnp.take_along_axis(
                        src_rows, kvec, axis=0, mode="promise_in_bounds")
                    src0 = base + lane_iota
                    dst = (r0 + k) * DIM
                    for j in range(DIM // LANES):
                        col = plsc.load_gather(table_v, [src0 + j * LANES])
                        rows_ref[pl.ds(dst + j * LANES, LANES)] = col
                return carry

            lax.fori_loop(0, CHUNK // UNROLL, rows_body, 0)

        def body(j, carry):
            g0 = j * NBUF
            for b in range(NBUF):
                idx_copy(g0 + b, b).wait()

                @pl.when(j > 0)
                def _(b=b):
                    out_copy(g0 + b - NBUF, b).wait()

                assemble(b)
                out_copy(g0 + b, b).start()

                @pl.when(j < n_iters - 1)
                def _(b=b):
                    idx_copy(g0 + b + NBUF, b).start()
            return carry

        lax.fori_loop(0, n_iters, body, 0)
        for b in range(NBUF):
            out_copy(n_chunks - NBUF + b, b).wait()

    return sc_gather


def kernel(x, emb_table, W, b):
    batch, hist = x.shape
    table_proj = _project_table(emb_table, W, b)
    flat_idx = x.reshape(-1)
    gather = _make_sc_gather(batch * hist)
    out = gather(table_proj.reshape(-1), flat_idx)
    return out.reshape(batch, hist, DIM)


# speedup vs baseline: 21.7936x; 4.3053x over previous
import functools

import jax
import jax.numpy as jnp
from jax import lax
from jax.experimental import pallas as pl
from jax.experimental.pallas import tpu as pltpu
from jax.experimental.pallas import tpu_sc as plsc

DIM = 128
VOCAB = 100
CHUNK = 128
NBUF = 4
UNROLL = 16
LANES = 16


def _project_body(emb_ref, w_ref, b_ref, out_ref):
    out_ref[...] = lax.dot_general(
        emb_ref[...], w_ref[...],
        dimension_numbers=(((1,), (1,)), ((), ())),
        preferred_element_type=jnp.float32,
    ) + b_ref[...]


def _project_table(emb_table, W, b):
    return pl.pallas_call(
        _project_body,
        out_shape=jax.ShapeDtypeStruct((VOCAB, DIM), jnp.float32),
    )(emb_table, W, b.reshape(1, DIM))


def _make_sc_gather(n_rows):
    info = plsc.get_sparse_core_info()
    nc, ns = info.num_cores, info.num_subcores
    nw = nc * ns
    assert n_rows % (nw * CHUNK * NBUF) == 0
    per_w = n_rows // nw
    n_chunks = per_w // CHUNK
    n_iters = n_chunks // NBUF
    mesh = plsc.VectorSubcoreMesh(core_axis_name="c", subcore_axis_name="s")

    scratch = (
        [pltpu.VMEM_SHARED((VOCAB, DIM), jnp.float32)]
        + [pltpu.VMEM((CHUNK,), jnp.int32) for _ in range(NBUF)]
        + [pltpu.VMEM((CHUNK, DIM), jnp.float32) for _ in range(NBUF)]
        + [pltpu.SemaphoreType.DMA] * (3 * NBUF + 1)
    )

    @functools.partial(
        pl.kernel,
        mesh=mesh,
        out_type=jax.ShapeDtypeStruct((n_rows, DIM), jnp.float32),
        scratch_types=scratch,
    )
    def sc_gather(table_hbm, idx_hbm, out_hbm, *bufs):
        table_sh = bufs[0]
        idx_v = bufs[1:1 + NBUF]
        rows_v = bufs[1 + NBUF:1 + 2 * NBUF]
        idx_sem = bufs[1 + 2 * NBUF:1 + 3 * NBUF]
        gat_sem = bufs[1 + 3 * NBUF:1 + 4 * NBUF]
        out_sem = bufs[1 + 4 * NBUF:1 + 5 * NBUF]
        tab_sem = bufs[1 + 5 * NBUF]
        sid = lax.axis_index("s")
        wid = sid * nc + lax.axis_index("c")
        base = wid * per_w

        def idx_copy(g, b):
            off = base + g * CHUNK
            return pltpu.make_async_copy(
                idx_hbm.at[pl.ds(off, CHUNK)], idx_v[b], idx_sem[b])

        def gat_copy(b):
            return pltpu.make_async_copy(
                table_sh.at[idx_v[b]], rows_v[b], gat_sem[b])

        def out_copy(g, b):
            off = base + g * CHUNK
            return pltpu.make_async_copy(
                rows_v[b], out_hbm.at[pl.ds(off, CHUNK)], out_sem[b])

        for b in range(NBUF):
            idx_copy(b, b).start()

        @pl.when(sid == 0)
        def _():
            pltpu.make_async_copy(table_hbm, table_sh, tab_sem).start()
            pltpu.make_async_copy(table_hbm, table_sh, tab_sem).wait()

        plsc.subcore_barrier()

        def body(j, carry):
            g0 = j * NBUF
            for b in range(NBUF):
                idx_copy(g0 + b, b).wait()

                @pl.when(j > 0)
                def _(b=b):
                    out_copy(g0 + b - NBUF, b).wait()

                gat_copy(b).start()
            for b in range(NBUF):
                gat_copy(b).wait()
                out_copy(g0 + b, b).start()

                @pl.when(j < n_iters - 1)
                def _(b=b):
                    idx_copy(g0 + b + NBUF, b).start()
            return carry

        lax.fori_loop(0, n_iters, body, 0)
        for b in range(NBUF):
            out_copy(n_chunks - NBUF + b, b).wait()

    return sc_gather


def kernel(x, emb_table, W, b):
    batch, hist = x.shape
    table_proj = _project_table(emb_table, W, b)
    flat_idx = x.reshape(-1)
    gather = _make_sc_gather(batch * hist)
    out = gather(table_proj, flat_idx)
    return out.reshape(batch, hist, DIM)
